# baseline (speedup 1.0000x reference)
import functools

import jax
import jax.numpy as jnp
from jax import lax
from jax.experimental import pallas as pl
from jax.experimental.pallas import tpu as pltpu
from jax.experimental.pallas import tpu_sc as plsc

E = 320000
D = 128
NW = 3
NC = 5
NCPAD = 8
NWORKERS = 32
EPW = E // NWORKERS
CHUNK = 80
NCHUNKS = EPW // CHUNK
NGROUPS = CHUNK // 16


def _sc_kernel(u_hbm, v_hbm, ui_hbm, vi_hbm, w_hbm, s_hbm, out_hbm,
               uidx, vidx, urows, vrows, outb, wbuf, sbuf,
               gsem, osem):
    core = lax.axis_index("c")
    sub = lax.axis_index("s")
    wid = sub * 2 + core
    base0 = wid * EPW

    pltpu.sync_copy(w_hbm, wbuf)
    pltpu.sync_copy(s_hbm, sbuf)
    pltpu.sync_copy(ui_hbm.at[pl.ds(base0, EPW)], uidx)
    pltpu.sync_copy(vi_hbm.at[pl.ds(base0, EPW)], vidx)

    lanes = lax.iota(jnp.int32, 16)

    mix = [[sbuf[pl.ds((w * 8 + c) * 16, 16)] for c in range(NC)]
           for w in range(NW)]

    def gather_copies(j, b):
        return (
            pltpu.make_async_copy(
                u_hbm.at[uidx.at[pl.ds(j * CHUNK, CHUNK)]], urows.at[b],
                gsem.at[b, 0]),
            pltpu.make_async_copy(
                v_hbm.at[vidx.at[pl.ds(j * CHUNK, CHUNK)]], vrows.at[b],
                gsem.at[b, 1]),
        )

    def out_copy(j, b):
        base = base0 + j * CHUNK
        return pltpu.make_async_copy(
            outb.at[b], out_hbm.at[pl.ds(base, CHUNK)], osem.at[b])

    def issue_gather(j, b):
        for c in gather_copies(j, b):
            c.start()

    def wait_gather(j, b):
        for c in gather_copies(j, b):
            c.wait()

    def compute_chunk(b):
        zeros = jnp.zeros((16,), jnp.float32)
        accs0 = tuple(zeros for _ in range(NGROUPS * NW))

        def d_body(t, accs):
            accs = list(accs)
            dvec = (lanes + t) & 63
            wa = [wbuf[pl.ds(t * 128 + w * 16, 16)] for w in range(NW)]
            wb = [wbuf[pl.ds(t * 128 + 64 + w * 16, 16)] for w in range(NW)]
            for g in range(NGROUPS):
                evec = lanes + g * 16
                pu = plsc.load_gather(urows.at[b], [evec, dvec])
                pv = plsc.load_gather(vrows.at[b], [evec, dvec])
                pbf = (plsc.bitcast(pu, jnp.bfloat16)
                       * plsc.bitcast(pv, jnp.bfloat16))
                pa, pb = plsc.unpack(pbf, format=plsc.PackFormat.INTERLEAVED)
                for w in range(NW):
                    accs[g * NW + w] = (accs[g * NW + w]
                                        + pa * wa[w] + pb * wb[w])
            return tuple(accs)

        accs = plsc.parallel_loop(0, D // 2, unroll=4, carry=accs0)(
            lambda t, accs: d_body(t, accs))

        for g in range(NGROUPS):
            evec = lanes + g * 16
            for c in range(NC):
                o = (accs[g * NW + 0] * mix[0][c]
                     + accs[g * NW + 1] * mix[1][c]
                     + accs[g * NW + 2] * mix[2][c])
                plsc.store_scatter(outb.at[b],
                                   [evec, jnp.full((16,), c, jnp.int32)], o)

    issue_gather(0, 0)

    def outer(jo, carry):
        for b in range(2):
            j = 2 * jo + b
            wait_gather(j, b)
            issue_gather(j + 1, 1 - b)

            @pl.when(jo > 0)
            def _():
                out_copy(j - 2, b).wait()

            compute_chunk(b)
            out_copy(j, b).start()
        return carry

    lax.fori_loop(0, NCHUNKS // 2, outer, 0)

    j = NCHUNKS - 1
    wait_gather(j, 0)
    out_copy(j - 2, 0).wait()
    compute_chunk(0)
    out_copy(j, 0).start()
    out_copy(j - 1, 1).wait()
    out_copy(j, 0).wait()


def kernel(u_features, v_features, u_indices, v_indices, weights, weights_scalars):
    mesh = plsc.VectorSubcoreMesh(core_axis_name="c", subcore_axis_name="s")
    w_pad = jnp.zeros((4, D), jnp.float32).at[:NW].set(weights)
    wt = w_pad.T
    t_idx = (jnp.arange(D // 2)[:, None] + jnp.arange(16)[None, :]) % (D // 2)
    w_a = wt[2 * t_idx]
    w_b = wt[2 * t_idx + 1]
    w_bcast = jnp.stack([w_a, w_b], axis=1).transpose(0, 1, 3, 2).reshape(-1)
    s_pad = jnp.zeros((4, 8), jnp.float32).at[:NW, :NC].set(weights_scalars)
    s_bcast = jnp.broadcast_to(s_pad[:, :, None], (4, 8, 16)).reshape(-1)
    ui = u_indices.astype(jnp.int32)
    vi = v_indices.astype(jnp.int32)
    u32 = jax.lax.bitcast_convert_type(
        u_features.astype(jnp.bfloat16).reshape(-1, D // 2, 2), jnp.int32)
    v32 = jax.lax.bitcast_convert_type(
        v_features.astype(jnp.bfloat16).reshape(-1, D // 2, 2), jnp.int32)
    u32 = jnp.concatenate([u32, jnp.zeros_like(u32)], axis=1)
    v32 = jnp.concatenate([v32, jnp.zeros_like(v32)], axis=1)

    run = functools.partial(
        pl.kernel,
        mesh=mesh,
        compiler_params=pltpu.CompilerParams(needs_layout_passes=False, disable_bounds_checks=True),
        out_type=jax.ShapeDtypeStruct((E, NCPAD), jnp.float32),
        scratch_types=[
            pltpu.VMEM((EPW,), jnp.int32),
            pltpu.VMEM((EPW,), jnp.int32),
            pltpu.VMEM((2, CHUNK, D), jnp.int32),
            pltpu.VMEM((2, CHUNK, D), jnp.int32),
            pltpu.VMEM((2, CHUNK, NCPAD), jnp.float32),
            pltpu.VMEM((D // 2 * 2 * 4 * 16,), jnp.float32),
            pltpu.VMEM((4 * 8 * 16,), jnp.float32),
            pltpu.SemaphoreType.DMA((2, 2)),
            pltpu.SemaphoreType.DMA((2,)),
        ],
    )(_sc_kernel)
    out = run(u32, v32, ui, vi, w_bcast, s_bcast)
    return out[:, :NC]


# speedup vs baseline: 1.2673x; 1.2673x over previous
import functools

import jax
import jax.numpy as jnp
from jax import lax
from jax.experimental import pallas as pl
from jax.experimental.pallas import tpu as pltpu
from jax.experimental.pallas import tpu_sc as plsc

E = 320000
D = 128
NW = 3
NC = 5
NCPAD = 8
NWORKERS = 32
EPW = E // NWORKERS
CHUNK = 80
NCHUNKS = EPW // CHUNK
NGROUPS = CHUNK // 16


def _sc_kernel(u_hbm, v_hbm, ui_hbm, vi_hbm, w_hbm, s_hbm, out_hbm,
               uidx, vidx, urows, vrows, outb, wbuf, sbuf,
               gsem, osem):
    core = lax.axis_index("c")
    sub = lax.axis_index("s")
    wid = sub * 2 + core
    base0 = wid * EPW

    pltpu.sync_copy(w_hbm, wbuf)
    pltpu.sync_copy(s_hbm, sbuf)
    pltpu.sync_copy(ui_hbm.at[pl.ds(base0, EPW)], uidx)
    pltpu.sync_copy(vi_hbm.at[pl.ds(base0, EPW)], vidx)

    lanes = lax.iota(jnp.int32, 16)

    mix = [[sbuf[pl.ds((w * 8 + c) * 16, 16)] for c in range(NC)]
           for w in range(NW)]

    def gather_copies(j, b):
        return (
            pltpu.make_async_copy(
                u_hbm.at[uidx.at[pl.ds(j * CHUNK, CHUNK)]], urows.at[b],
                gsem.at[b, 0]),
            pltpu.make_async_copy(
                v_hbm.at[vidx.at[pl.ds(j * CHUNK, CHUNK)]], vrows.at[b],
                gsem.at[b, 1]),
        )

    def out_copy(j, b):
        base = base0 + j * CHUNK
        return pltpu.make_async_copy(
            outb.at[b], out_hbm.at[pl.ds(base, CHUNK)], osem.at[b])

    def issue_gather(j, b):
        for c in gather_copies(j, b):
            c.start()

    def wait_gather(j, b):
        for c in gather_copies(j, b):
            c.wait()

    def compute_chunk(b):
        zeros = jnp.zeros((16,), jnp.float32)
        accs0 = tuple(zeros for _ in range(NGROUPS * NW))

        def d_body(t, accs):
            accs = list(accs)
            dvec = (lanes + t) & 63
            wa = [wbuf[pl.ds(t * 128 + w * 16, 16)] for w in range(NW)]
            wb = [wbuf[pl.ds(t * 128 + 64 + w * 16, 16)] for w in range(NW)]
            for g in range(NGROUPS):
                evec = lanes + g * 16
                pu = plsc.load_gather(urows.at[b], [evec, dvec])
                pv = plsc.load_gather(vrows.at[b], [evec, dvec])
                pbf = (plsc.bitcast(pu, jnp.bfloat16)
                       * plsc.bitcast(pv, jnp.bfloat16))
                pa, pb = plsc.unpack(pbf, format=plsc.PackFormat.INTERLEAVED)
                for w in range(NW):
                    accs[g * NW + w] = (accs[g * NW + w]
                                        + pa * wa[w] + pb * wb[w])
            return tuple(accs)

        accs = plsc.parallel_loop(0, D // 2, unroll=1, carry=accs0)(
            lambda t, accs: d_body(t, accs))

        for g in range(NGROUPS):
            evec = lanes + g * 16
            for c in range(NC):
                o = (accs[g * NW + 0] * mix[0][c]
                     + accs[g * NW + 1] * mix[1][c]
                     + accs[g * NW + 2] * mix[2][c])
                plsc.store_scatter(outb.at[b],
                                   [evec, jnp.full((16,), c, jnp.int32)], o)

    issue_gather(0, 0)

    def outer(jo, carry):
        for b in range(2):
            j = 2 * jo + b
            wait_gather(j, b)
            issue_gather(j + 1, 1 - b)

            @pl.when(jo > 0)
            def _():
                out_copy(j - 2, b).wait()

            compute_chunk(b)
            out_copy(j, b).start()
        return carry

    lax.fori_loop(0, NCHUNKS // 2, outer, 0)

    j = NCHUNKS - 1
    wait_gather(j, 0)
    out_copy(j - 2, 0).wait()
    compute_chunk(0)
    out_copy(j, 0).start()
    out_copy(j - 1, 1).wait()
    out_copy(j, 0).wait()


def kernel(u_features, v_features, u_indices, v_indices, weights, weights_scalars):
    mesh = plsc.VectorSubcoreMesh(core_axis_name="c", subcore_axis_name="s")
    w_pad = jnp.zeros((4, D), jnp.float32).at[:NW].set(weights)
    wt = w_pad.T
    t_idx = (jnp.arange(D // 2)[:, None] + jnp.arange(16)[None, :]) % (D // 2)
    w_a = wt[2 * t_idx]
    w_b = wt[2 * t_idx + 1]
    w_bcast = jnp.stack([w_a, w_b], axis=1).transpose(0, 1, 3, 2).reshape(-1)
    s_pad = jnp.zeros((4, 8), jnp.float32).at[:NW, :NC].set(weights_scalars)
    s_bcast = jnp.broadcast_to(s_pad[:, :, None], (4, 8, 16)).reshape(-1)
    ui = u_indices.astype(jnp.int32)
    vi = v_indices.astype(jnp.int32)
    u32 = jax.lax.bitcast_convert_type(
        u_features.astype(jnp.bfloat16).reshape(-1, D // 2, 2), jnp.int32)
    v32 = jax.lax.bitcast_convert_type(
        v_features.astype(jnp.bfloat16).reshape(-1, D // 2, 2), jnp.int32)
    u32 = jnp.concatenate([u32, jnp.zeros_like(u32)], axis=1)
    v32 = jnp.concatenate([v32, jnp.zeros_like(v32)], axis=1)

    run = functools.partial(
        pl.kernel,
        mesh=mesh,
        compiler_params=pltpu.CompilerParams(needs_layout_passes=False, disable_bounds_checks=True),
        out_type=jax.ShapeDtypeStruct((E, NCPAD), jnp.float32),
        scratch_types=[
            pltpu.VMEM((EPW,), jnp.int32),
            pltpu.VMEM((EPW,), jnp.int32),
            pltpu.VMEM((2, CHUNK, D), jnp.int32),
            pltpu.VMEM((2, CHUNK, D), jnp.int32),
            pltpu.VMEM((2, CHUNK, NCPAD), jnp.float32),
            pltpu.VMEM((D // 2 * 2 * 4 * 16,), jnp.float32),
            pltpu.VMEM((4 * 8 * 16,), jnp.float32),
            pltpu.SemaphoreType.DMA((2, 2)),
            pltpu.SemaphoreType.DMA((2,)),
        ],
    )(_sc_kernel)
    out = run(u32, v32, ui, vi, w_bcast, s_bcast)
    return out[:, :NC]


# speedup vs baseline: 1.3439x; 1.0604x over previous
import functools

import jax
import jax.numpy as jnp
from jax import lax
from jax.experimental import pallas as pl
from jax.experimental.pallas import tpu as pltpu
from jax.experimental.pallas import tpu_sc as plsc

E = 320000
D = 128
NW = 3
NC = 5
NCPAD = 8
NWORKERS = 32
EPW = E // NWORKERS
CHUNK = 80
NCHUNKS = EPW // CHUNK
NGROUPS = CHUNK // 16


def _sc_kernel(u_hbm, v_hbm, ui_hbm, vi_hbm, w_hbm, s_hbm, out_hbm,
               uidx, vidx, urows, vrows, outb, wbuf, sbuf,
               gsem, osem):
    core = lax.axis_index("c")
    sub = lax.axis_index("s")
    wid = sub * 2 + core
    base0 = wid * EPW

    pltpu.sync_copy(w_hbm, wbuf)
    pltpu.sync_copy(s_hbm, sbuf)
    pltpu.sync_copy(ui_hbm.at[pl.ds(base0, EPW)], uidx)
    pltpu.sync_copy(vi_hbm.at[pl.ds(base0, EPW)], vidx)

    lanes = lax.iota(jnp.int32, 16)

    mix = [[sbuf[pl.ds((w * 8 + c) * 16, 16)] for c in range(NC)]
           for w in range(NW)]

    def gather_copies(j, b):
        return (
            pltpu.make_async_copy(
                u_hbm.at[uidx.at[pl.ds(j * CHUNK, CHUNK)]], urows.at[b],
                gsem.at[b, 0]),
            pltpu.make_async_copy(
                v_hbm.at[vidx.at[pl.ds(j * CHUNK, CHUNK)]], vrows.at[b],
                gsem.at[b, 1]),
        )

    def out_copy(j, b):
        cid = wid * NCHUNKS + j
        return pltpu.make_async_copy(outb.at[b], out_hbm.at[cid], osem.at[b])

    def issue_gather(j, b):
        for c in gather_copies(j, b):
            c.start()

    def wait_gather(j, b):
        for c in gather_copies(j, b):
            c.wait()

    def compute_chunk(b):
        zeros = jnp.zeros((16,), jnp.float32)
        accs0 = tuple(zeros for _ in range(NGROUPS * NW))

        def d_body(t, accs):
            accs = list(accs)
            dvec = (lanes + t) & 63
            wa = [wbuf[pl.ds(t * 128 + w * 16, 16)] for w in range(NW)]
            wb = [wbuf[pl.ds(t * 128 + 64 + w * 16, 16)] for w in range(NW)]
            for g in range(NGROUPS):
                evec = lanes + g * 16
                pu = plsc.load_gather(urows.at[b], [evec, dvec])
                pv = plsc.load_gather(vrows.at[b], [evec, dvec])
                pbf = (plsc.bitcast(pu, jnp.bfloat16)
                       * plsc.bitcast(pv, jnp.bfloat16))
                pa, pb = plsc.unpack(pbf, format=plsc.PackFormat.INTERLEAVED)
                for w in range(NW):
                    accs[g * NW + w] = (accs[g * NW + w]
                                        + pa * wa[w] + pb * wb[w])
            return tuple(accs)

        accs = plsc.parallel_loop(0, D // 2, unroll=1, carry=accs0)(
            lambda t, accs: d_body(t, accs))

        for g in range(NGROUPS):
            for c in range(NC):
                o = (accs[g * NW + 0] * mix[0][c]
                     + accs[g * NW + 1] * mix[1][c]
                     + accs[g * NW + 2] * mix[2][c])
                outb.at[b][c, pl.ds(g * 16, 16)] = o

    issue_gather(0, 0)

    def outer(jo, carry):
        for b in range(2):
            j = 2 * jo + b
            wait_gather(j, b)
            issue_gather(j + 1, 1 - b)

            @pl.when(jo > 0)
            def _():
                out_copy(j - 2, b).wait()

            compute_chunk(b)
            out_copy(j, b).start()
        return carry

    lax.fori_loop(0, NCHUNKS // 2, outer, 0)

    j = NCHUNKS - 1
    wait_gather(j, 0)
    out_copy(j - 2, 0).wait()
    compute_chunk(0)
    out_copy(j, 0).start()
    out_copy(j - 1, 1).wait()
    out_copy(j, 0).wait()


def kernel(u_features, v_features, u_indices, v_indices, weights, weights_scalars):
    mesh = plsc.VectorSubcoreMesh(core_axis_name="c", subcore_axis_name="s")
    w_pad = jnp.zeros((4, D), jnp.float32).at[:NW].set(weights)
    wt = w_pad.T
    t_idx = (jnp.arange(D // 2)[:, None] + jnp.arange(16)[None, :]) % (D // 2)
    w_a = wt[2 * t_idx]
    w_b = wt[2 * t_idx + 1]
    w_bcast = jnp.stack([w_a, w_b], axis=1).transpose(0, 1, 3, 2).reshape(-1)
    s_pad = jnp.zeros((4, 8), jnp.float32).at[:NW, :NC].set(weights_scalars)
    s_bcast = jnp.broadcast_to(s_pad[:, :, None], (4, 8, 16)).reshape(-1)
    ui = u_indices.astype(jnp.int32)
    vi = v_indices.astype(jnp.int32)
    u32 = jax.lax.bitcast_convert_type(
        u_features.astype(jnp.bfloat16).reshape(-1, D // 2, 2), jnp.int32)
    v32 = jax.lax.bitcast_convert_type(
        v_features.astype(jnp.bfloat16).reshape(-1, D // 2, 2), jnp.int32)
    u32 = jnp.concatenate([u32, jnp.zeros_like(u32)], axis=1)
    v32 = jnp.concatenate([v32, jnp.zeros_like(v32)], axis=1)

    run = functools.partial(
        pl.kernel,
        mesh=mesh,
        compiler_params=pltpu.CompilerParams(needs_layout_passes=False, disable_bounds_checks=True),
        out_type=jax.ShapeDtypeStruct((E // CHUNK, NCPAD, CHUNK), jnp.float32),
        scratch_types=[
            pltpu.VMEM((EPW,), jnp.int32),
            pltpu.VMEM((EPW,), jnp.int32),
            pltpu.VMEM((2, CHUNK, D), jnp.int32),
            pltpu.VMEM((2, CHUNK, D), jnp.int32),
            pltpu.VMEM((2, NCPAD, CHUNK), jnp.float32),
            pltpu.VMEM((D // 2 * 2 * 4 * 16,), jnp.float32),
            pltpu.VMEM((4 * 8 * 16,), jnp.float32),
            pltpu.SemaphoreType.DMA((2, 2)),
            pltpu.SemaphoreType.DMA((2,)),
        ],
    )(_sc_kernel)
    out = run(u32, v32, ui, vi, w_bcast, s_bcast)
    return out.transpose(0, 2, 1).reshape(E, NCPAD)[:, :NC]
